# baseline (speedup 1.0000x reference)
import functools

import jax
import jax.numpy as jnp
from jax import lax
from jax.experimental import pallas as pl
from jax.experimental.pallas import tpu as pltpu
from jax.experimental.pallas import tpu_sc as plsc

NC, NS, LANES = 2, 16, 16
NW = NC * NS
VOCAB, D = 100000, 64
B, L = 4096, 200
TOK = B * L
PER_W = TOK // NW
SUPER = 512
IDXW = 128
KG = SUPER // IDXW
NSUPER = PER_W // SUPER
IDX_ROWS_PER_W = PER_W // IDXW

_mesh = plsc.VectorSubcoreMesh(
    core_axis_name="c", subcore_axis_name="s", num_cores=NC, num_subcores=NS
)


def _sc_body(table_hbm, idx_hbm, ratio_hbm, out_hbm, idx_v, ratio_v,
             r0, r1, r2, r3, r4, r5, r6, r7, gs0, gs1, os0, os1):
    wid = lax.axis_index("s") * NC + lax.axis_index("c")
    rows = ((r0, r1, r2, r3), (r4, r5, r6, r7))
    gsem = (gs0, gs1)
    osem = (os0, os1)

    pltpu.sync_copy(idx_hbm.at[pl.ds(wid * IDX_ROWS_PER_W, IDX_ROWS_PER_W)], idx_v)
    pltpu.sync_copy(ratio_hbm.at[pl.ds(wid * PER_W, PER_W)], ratio_v)

    def fire_gathers(s, b):
        for j in range(KG):
            pltpu.async_copy(
                table_hbm.at[idx_v.at[s * KG + j]],
                rows[b][j],
                gsem[b],
            )

    def wait_gathers(b):
        for j in range(KG):
            pltpu.make_async_copy(
                table_hbm.at[idx_v.at[j]],
                rows[b][j],
                gsem[b],
            ).wait()

    def wait_outs(b):
        for j in range(KG):
            pltpu.make_async_copy(
                rows[b][j],
                out_hbm.at[pl.ds(0, IDXW)],
                osem[b],
            ).wait()

    def slot(s, b):
        @pl.when(s >= 1)
        def _():
            wait_outs(1 - b)

        @pl.when(s + 1 < NSUPER)
        def _():
            fire_gathers(s + 1, 1 - b)

        wait_gathers(b)
        for i in range(KG):
            def mul_body(t, c):
                rv = ratio_v[pl.ds(s * SUPER + i * IDXW + t * LANES, LANES)] * 8.0
                for k in range(LANES):
                    rvec = jnp.full((LANES,), rv[k], jnp.float32)
                    row = t * LANES + k
                    for j in range(D // LANES):
                        sl = pl.ds(j * LANES, LANES)
                        rows[b][i][row, sl] = rows[b][i][row, sl] * rvec
                return c

            lax.fori_loop(0, IDXW // LANES, mul_body, 0)
            pltpu.async_copy(
                rows[b][i],
                out_hbm.at[pl.ds(wid * PER_W + s * SUPER + i * IDXW, IDXW)],
                osem[b],
            )

    fire_gathers(0, 0)

    def loop_body(t, c):
        slot(2 * t, 0)
        slot(2 * t + 1, 1)
        return c

    lax.fori_loop(0, NSUPER // 2, loop_body, 0)

    wait_outs(1)


_sc_call = functools.partial(
    pl.kernel,
    out_type=jax.ShapeDtypeStruct((TOK, D), jnp.float32),
    mesh=_mesh,
    compiler_params=pltpu.CompilerParams(use_tc_tiling_on_sc=False),
    scratch_types=[
        pltpu.VMEM((IDX_ROWS_PER_W, IDXW), jnp.int32),
        pltpu.VMEM((PER_W,), jnp.float32),
        pltpu.VMEM((IDXW, D), jnp.float32),
        pltpu.VMEM((IDXW, D), jnp.float32),
        pltpu.VMEM((IDXW, D), jnp.float32),
        pltpu.VMEM((IDXW, D), jnp.float32),
        pltpu.VMEM((IDXW, D), jnp.float32),
        pltpu.VMEM((IDXW, D), jnp.float32),
        pltpu.VMEM((IDXW, D), jnp.float32),
        pltpu.VMEM((IDXW, D), jnp.float32),
        pltpu.SemaphoreType.DMA,
        pltpu.SemaphoreType.DMA,
        pltpu.SemaphoreType.DMA,
        pltpu.SemaphoreType.DMA,
    ],
)(_sc_body)


def kernel(x, table):
    words = x[:, 0, :].reshape(TOK).astype(jnp.int32)
    ratio = x[:, 1, :].reshape(TOK)
    idx2d = words.reshape(TOK // IDXW, IDXW)
    out = _sc_call(table, idx2d, ratio)
    return out.reshape(B, L, D)
